# Initial kernel scaffold; baseline (speedup 1.0000x reference)
import functools

import jax
import jax.numpy as jnp
from jax import lax
from jax.experimental import pallas as pl
from jax.experimental.pallas import tpu as pltpu
from jax.experimental.pallas import tpu_sc as plsc

_EPS = 1e-5
_GATHER_WINDOW = 8


def _sc_gather(word_table, idx):
    n = idx.shape[0]
    h = word_table.shape[1]
    w = _GATHER_WINDOW
    mesh = plsc.VectorSubcoreMesh(core_axis_name="core", subcore_axis_name="subcore")
    idx2 = idx.reshape(1, n)

    @functools.partial(
        pl.kernel,
        out_type=jax.ShapeDtypeStruct((n, h), jnp.float32),
        mesh=mesh,
    )
    def gather_kernel(x_hbm, i_hbm, o_hbm):
        def body(i_vmem, o_vmem):
            pltpu.sync_copy(x_hbm.at[i_vmem.at[0]], o_vmem)

        pltpu.emit_pipeline(
            body,
            grid=(n // w,),
            in_specs=[pl.BlockSpec((1, w), index_map=lambda i: (0, i))],
            out_specs=[pl.BlockSpec((w, h), index_map=lambda i: (i, 0))],
            core_axis_name=("core", "subcore"),
            dimension_semantics=(pltpu.PARALLEL,),
        )(i_hbm, o_hbm)

    return gather_kernel(word_table, idx2)


def _ln_body(g_ref, p_ref, t_ref, tt_ref, gam_ref, bet_ref, o_ref):
    x = g_ref[...] + p_ref[...]
    t0 = t_ref[0, :][None, :]
    dt = (t_ref[1, :] - t_ref[0, :])[None, :]
    x = x + t0 + tt_ref[...] * dt
    mean = jnp.mean(x, axis=-1, keepdims=True)
    xc = x - mean
    var = jnp.mean(xc * xc, axis=-1, keepdims=True)
    y = xc * lax.rsqrt(var + _EPS)
    o_ref[...] = y * gam_ref[...] + bet_ref[...]


def _tc_add_ln(gathered, pos_table, tt_f, type_table, gamma2, beta2, tokens_per_block):
    n, h = gathered.shape
    s = pos_table.shape[0]
    t = tokens_per_block
    s_blocks = s // t
    b = n // s

    return pl.pallas_call(
        _ln_body,
        grid=(s_blocks, b),
        in_specs=[
            pl.BlockSpec((t, h), lambda i, j: (j * s_blocks + i, 0)),
            pl.BlockSpec((t, h), lambda i, j: (i, 0)),
            pl.BlockSpec((2, h), lambda i, j: (0, 0)),
            pl.BlockSpec((t, 1), lambda i, j: (j * s_blocks + i, 0)),
            pl.BlockSpec((1, h), lambda i, j: (0, 0)),
            pl.BlockSpec((1, h), lambda i, j: (0, 0)),
        ],
        out_specs=pl.BlockSpec((t, h), lambda i, j: (j * s_blocks + i, 0)),
        out_shape=jax.ShapeDtypeStruct((n, h), jnp.float32),
    )(gathered, pos_table, type_table, tt_f, gamma2, beta2)


def kernel(input_ids, token_type_ids, word_table, pos_table, type_table, gamma, beta):
    batch, seq = input_ids.shape
    h = word_table.shape[1]
    idx = input_ids.reshape(-1).astype(jnp.int32)
    tt_f = token_type_ids.reshape(-1, 1).astype(jnp.float32)
    gathered = _sc_gather(word_table, idx)
    out = _tc_add_ln(
        gathered,
        pos_table,
        tt_f,
        type_table,
        gamma.reshape(1, h),
        beta.reshape(1, h),
        tokens_per_block=256,
    )
    return out.reshape(batch, seq, h)


# speedup vs baseline: 1.8416x; 1.8416x over previous
import functools

import jax
import jax.numpy as jnp
from jax import lax
from jax.experimental import pallas as pl
from jax.experimental.pallas import tpu as pltpu
from jax.experimental.pallas import tpu_sc as plsc

_EPS = 1e-5
_GATHER_CHUNK = 16


def _sc_gather(word_table, idx):
    n = idx.shape[0]
    h = word_table.shape[1]
    mesh = plsc.VectorSubcoreMesh(core_axis_name="core", subcore_axis_name="subcore")
    num_workers = mesh.num_cores * mesh.num_subcores
    b_per_w = n // num_workers
    chunk = _GATHER_CHUNK
    nchunks = b_per_w // chunk

    @functools.partial(
        pl.kernel,
        out_type=jax.ShapeDtypeStruct((n, h), jnp.float32),
        mesh=mesh,
        scratch_types=[
            pltpu.VMEM((b_per_w,), jnp.int32),
            pltpu.VMEM((chunk, h), jnp.float32),
            pltpu.VMEM((chunk, h), jnp.float32),
            pltpu.SemaphoreType.DMA,
            pltpu.SemaphoreType.DMA,
        ],
    )
    def gather_kernel(x_hbm, i_hbm, o_hbm, idx_v, buf0, buf1, sem0, sem1):
        wid = lax.axis_index("subcore") * mesh.num_cores + lax.axis_index("core")
        base = wid * b_per_w
        pltpu.sync_copy(i_hbm.at[pl.ds(base, b_per_w)], idx_v)
        bufs = (buf0, buf1)
        sems = (sem0, sem1)

        def start(c):
            return pltpu.async_copy(
                x_hbm.at[idx_v.at[pl.ds(c * chunk, chunk)]],
                bufs[c % 2],
                sems[c % 2],
            )

        pending = start(0)
        for c in range(nchunks):
            nxt = start(c + 1) if c + 1 < nchunks else None
            pending.wait()
            pltpu.sync_copy(bufs[c % 2], o_hbm.at[pl.ds(base + c * chunk, chunk)])
            pending = nxt

    return gather_kernel(word_table, idx)


def _ln_body(g_ref, p_ref, t_ref, tt_ref, gam_ref, bet_ref, o_ref):
    x = g_ref[...] + p_ref[...]
    t0 = t_ref[0, :][None, :]
    dt = (t_ref[1, :] - t_ref[0, :])[None, :]
    x = x + t0 + tt_ref[...] * dt
    mean = jnp.mean(x, axis=-1, keepdims=True)
    xc = x - mean
    var = jnp.mean(xc * xc, axis=-1, keepdims=True)
    y = xc * lax.rsqrt(var + _EPS)
    o_ref[...] = y * gam_ref[...] + bet_ref[...]


def _tc_add_ln(gathered, pos_table, tt_f, type_table, gamma2, beta2, tokens_per_block):
    n, h = gathered.shape
    s = pos_table.shape[0]
    t = tokens_per_block
    s_blocks = s // t
    b = n // s

    return pl.pallas_call(
        _ln_body,
        grid=(s_blocks, b),
        in_specs=[
            pl.BlockSpec((t, h), lambda i, j: (j * s_blocks + i, 0)),
            pl.BlockSpec((t, h), lambda i, j: (i, 0)),
            pl.BlockSpec((2, h), lambda i, j: (0, 0)),
            pl.BlockSpec((t, 1), lambda i, j: (j * s_blocks + i, 0)),
            pl.BlockSpec((1, h), lambda i, j: (0, 0)),
            pl.BlockSpec((1, h), lambda i, j: (0, 0)),
        ],
        out_specs=pl.BlockSpec((t, h), lambda i, j: (j * s_blocks + i, 0)),
        out_shape=jax.ShapeDtypeStruct((n, h), jnp.float32),
    )(gathered, pos_table, type_table, tt_f, gamma2, beta2)


def kernel(input_ids, token_type_ids, word_table, pos_table, type_table, gamma, beta):
    batch, seq = input_ids.shape
    h = word_table.shape[1]
    idx = input_ids.reshape(-1).astype(jnp.int32)
    tt_f = token_type_ids.reshape(-1, 1).astype(jnp.float32)
    gathered = _sc_gather(word_table, idx)
    out = _tc_add_ln(
        gathered,
        pos_table,
        tt_f,
        type_table,
        gamma.reshape(1, h),
        beta.reshape(1, h),
        tokens_per_block=256,
    )
    return out.reshape(batch, seq, h)
